# baseline (speedup 1.0000x reference)
import functools

import jax
import jax.numpy as jnp
from jax import lax
from jax.experimental import pallas as pl
from jax.experimental.pallas import tpu as pltpu
from jax.experimental.pallas import tpu_sc as plsc

VOCAB = 1000000
EMBED_DIM = 64
BATCH = 4096
HIST = 200

NC, NS = 2, 16
NW = NC * NS
LANES = 16
CHUNK = 128
BLOCKS_PER_T = BATCH // CHUNK
N_CHUNKS = HIST * BLOCKS_PER_T
CH_PER_W = N_CHUNKS // NW
NBUF = 2
N_GROUPS = CH_PER_W // NBUF


def _gather_body(ids_t_hbm, table_hbm, out_hbm, idx0, idx1, rows0, rows1,
                 tr0, tr1, isem0, isem1, gsem0, gsem1, wsem0, wsem1):
    wid = lax.axis_index("s") * NC + lax.axis_index("c")
    c_base = wid * CH_PER_W
    idxs = (idx0, idx1)
    bufs = (rows0, rows1)
    trs = (tr0, tr1)
    isems = (isem0, isem1)
    gsems = (gsem0, gsem1)
    wsems = (wsem0, wsem1)

    def fire_idx(c, b):
        t = c // BLOCKS_PER_T
        b0 = (c % BLOCKS_PER_T) * CHUNK
        pltpu.async_copy(ids_t_hbm.at[t, pl.ds(b0, CHUNK)], idxs[b], isems[b])

    def fire_gather(b):
        pltpu.async_copy(table_hbm.at[idxs[b]], bufs[b], gsems[b])

    def drain_gather(b):
        pltpu.make_async_copy(table_hbm.at[pl.ds(0, CHUNK)], bufs[b],
                              gsems[b]).wait()

    lane = lax.iota(jnp.int32, LANES)
    diags = [(lane + k) % LANES for k in range(LANES)]

    def transpose(b):
        buf, tr = bufs[b], trs[b]

        def blk_step(cb, carry):
            cvec = lane + cb * LANES
            for e0 in range(0, EMBED_DIM, LANES):
                for k in range(LANES):
                    evec = diags[k] + e0
                    vals = plsc.load_gather(buf, [cvec, evec])
                    sb = evec >> 3
                    pos = ((evec & 7) << 7) + cvec
                    plsc.store_scatter(tr, [sb, pos], vals)
            return carry

        lax.fori_loop(0, CHUNK // LANES, blk_step, 0)

    def fire_wb(c, b):
        t = c // BLOCKS_PER_T
        vb = c % BLOCKS_PER_T
        pltpu.async_copy(trs[b], out_hbm.at[t, :, vb], wsems[b])

    def drain_wb(b):
        pltpu.make_async_copy(trs[b], out_hbm.at[0, :, 0], wsems[b]).wait()

    for b in range(NBUF):
        fire_idx(c_base + b, b)
    for b in range(NBUF):
        pltpu.make_async_copy(ids_t_hbm.at[0, pl.ds(0, CHUNK)], idxs[b],
                              isems[b]).wait()
        fire_gather(b)

    def group_step(k, carry):
        for b in range(NBUF):
            c = c_base + k * NBUF + b
            drain_gather(b)

            @pl.when(k < N_GROUPS - 1)
            def _():
                fire_idx(c + NBUF, b)

            @pl.when(k > 0)
            def _():
                drain_wb(b)

            transpose(b)
            fire_wb(c, b)

            @pl.when(k < N_GROUPS - 1)
            def _():
                pltpu.make_async_copy(ids_t_hbm.at[0, pl.ds(0, CHUNK)],
                                      idxs[b], isems[b]).wait()
                fire_gather(b)
        return carry

    lax.fori_loop(0, N_GROUPS, group_step, 0)
    for b in range(NBUF):
        drain_wb(b)


def kernel(input_ids, table):
    ids_t = input_ids.T.astype(jnp.int32)
    mesh = plsc.VectorSubcoreMesh(core_axis_name="c", subcore_axis_name="s")
    run = functools.partial(
        pl.kernel,
        mesh=mesh,
        out_type=jax.ShapeDtypeStruct(
            (HIST, EMBED_DIM // 8, BLOCKS_PER_T, 8 * CHUNK), jnp.float32),
        scratch_types=[
            pltpu.VMEM((CHUNK,), jnp.int32),
            pltpu.VMEM((CHUNK,), jnp.int32),
            pltpu.VMEM((CHUNK, EMBED_DIM), jnp.float32),
            pltpu.VMEM((CHUNK, EMBED_DIM), jnp.float32),
            pltpu.VMEM((EMBED_DIM // 8, 8 * CHUNK), jnp.float32),
            pltpu.VMEM((EMBED_DIM // 8, 8 * CHUNK), jnp.float32),
            pltpu.SemaphoreType.DMA,
            pltpu.SemaphoreType.DMA,
            pltpu.SemaphoreType.DMA,
            pltpu.SemaphoreType.DMA,
            pltpu.SemaphoreType.DMA,
            pltpu.SemaphoreType.DMA,
        ],
        compiler_params=pltpu.CompilerParams(use_tc_tiling_on_sc=False,
                                             needs_layout_passes=False),
    )(_gather_body)
    out4 = run(ids_t, table)
    out5 = out4.reshape(HIST, EMBED_DIM // 8, BLOCKS_PER_T, 8, CHUNK)
    return out5.transpose(2, 4, 0, 1, 3).reshape(BATCH, HIST, EMBED_DIM)
